# Initial kernel scaffold; baseline (speedup 1.0000x reference)
import jax
import jax.numpy as jnp
from jax.experimental import pallas as pl


def kernel(x, W):
    raise NotImplementedError("write your pallas kernel here")


# speedup vs baseline: 2.8555x; 2.8555x over previous
import jax
import jax.numpy as jnp
from jax.experimental import pallas as pl


def _add_row_kernel(x_ref, w_ref, o_ref):
    o_ref[...] = x_ref[...] + w_ref[...]


def kernel(x, W):
    B, L, D = x.shape
    patch = L // 3
    xr = x.reshape(B * 3, patch, D)
    Wr = W.reshape(3, 1, D)

    BL = 512
    grid = (B * 3, patch // BL)

    out = pl.pallas_call(
        _add_row_kernel,
        grid=grid,
        in_specs=[
            pl.BlockSpec((1, BL, D), lambda i, j: (i, j, 0)),
            pl.BlockSpec((1, 1, D), lambda i, j: (i % 3, 0, 0)),
        ],
        out_specs=pl.BlockSpec((1, BL, D), lambda i, j: (i, j, 0)),
        out_shape=jax.ShapeDtypeStruct((B * 3, patch, D), x.dtype),
    )(xr, Wr)
    return out.reshape(B, L, D)


# speedup vs baseline: 3.1041x; 1.0870x over previous
import jax
import jax.numpy as jnp
from jax.experimental import pallas as pl


def _add_row_kernel(x_ref, w_ref, o_ref):
    o_ref[...] = x_ref[...] + w_ref[...]


def kernel(x, W):
    B, L, D = x.shape
    patch = L // 3
    xr = x.reshape(B * 3, patch, D)
    Wr = W.reshape(3, 1, D)

    BL = 1024
    grid = (B * 3, patch // BL)

    out = pl.pallas_call(
        _add_row_kernel,
        grid=grid,
        in_specs=[
            pl.BlockSpec((1, BL, D), lambda i, j: (i, j, 0)),
            pl.BlockSpec((1, 1, D), lambda i, j: (i % 3, 0, 0)),
        ],
        out_specs=pl.BlockSpec((1, BL, D), lambda i, j: (i, j, 0)),
        out_shape=jax.ShapeDtypeStruct((B * 3, patch, D), x.dtype),
    )(xr, Wr)
    return out.reshape(B, L, D)
